# baseline (speedup 1.0000x reference)
import functools

import jax
import jax.numpy as jnp
from jax import lax
from jax.experimental import pallas as pl
from jax.experimental.pallas import tpu as pltpu
from jax.experimental.pallas import tpu_sc as plsc

N = 10000
E = 320000
F_IN = 128
H = 64
C = 40
CP = 48

NC = 2
NS = 16
NW = NC * NS
EW = E // NW
B = 80
NCHUNK = EW // B
RB = 80
NBUF = 5


def _mm1_body(x_ref, w_ref, b_ref, o_ref):
    o_ref[...] = (
        jnp.dot(x_ref[...], w_ref[...], preferred_element_type=jnp.float32)
        + b_ref[...]
    )


def _mm2_body(p0_ref, p1_ref, w_ref, b_ref, o_ref):
    h = jnp.maximum(p0_ref[...] + p1_ref[...], 0.0)
    o_ref[...] = (
        jnp.dot(h, w_ref[...], preferred_element_type=jnp.float32) + b_ref[...]
    )


def _lsm_body(q0_ref, q1_ref, o_ref):
    logits = (q0_ref[...] + q1_ref[...])[:, :C]
    m = jnp.max(logits, axis=1, keepdims=True)
    ex = jnp.exp(logits - m)
    lse = jnp.log(jnp.sum(ex, axis=1, keepdims=True)) + m
    o_ref[...] = logits - lse


def _make_spmm(D):
    mesh = plsc.VectorSubcoreMesh(core_axis_name="c", subcore_axis_name="s")

    @functools.partial(
        pl.kernel,
        out_type=[
            jax.ShapeDtypeStruct((N, D), jnp.float32),
            jax.ShapeDtypeStruct((N, D), jnp.float32),
        ],
        mesh=mesh,
        scratch_types=[
            pltpu.VMEM((EW,), jnp.int32),
            pltpu.VMEM((NCHUNK, B), jnp.int32),
            pltpu.VMEM((EW,), jnp.float32),
            [pltpu.VMEM((B, D), jnp.float32)] * NBUF,
            pltpu.VMEM_SHARED((N, D), jnp.float32),
            [pltpu.SemaphoreType.DMA] * NBUF,
            [pltpu.SemaphoreType.DMA] * NBUF,
        ],
        compiler_params=pltpu.CompilerParams(use_tc_tiling_on_sc=False),
    )
    def spmm(h_hbm, src_hbm, dst_hbm, w_hbm, p0_hbm, p1_hbm,
             srcv, dstv, wv, rows, acc, gsem, ssem):
        cid = lax.axis_index("c")
        sid = lax.axis_index("s")
        wid = sid * NC + cid

        zero16 = jnp.zeros((16,), jnp.float32)
        for e in range(B):
            for j in range(D // 16):
                rows[0][e, pl.ds(j * 16, 16)] = zero16
        for i in range(8):
            cz = sid * 8 + i

            @pl.when(cz < NCHUNK)
            def _():
                pltpu.sync_copy(rows[0], acc.at[pl.ds(cz * RB, RB)])

        pltpu.sync_copy(src_hbm.at[wid], srcv)
        pltpu.sync_copy(dst_hbm.at[wid], dstv)
        pltpu.sync_copy(w_hbm.at[wid], wv)

        plsc.subcore_barrier()

        def gather(c, i):
            return pltpu.make_async_copy(
                h_hbm.at[srcv.at[pl.ds(c * B, B)]], rows[i], gsem[i])

        def scatter(c, i):
            return pltpu.make_async_copy(rows[i], acc.at[dstv.at[c]], ssem[i])

        def scale(c, i):
            for g in range(B // 16):
                wvec = wv[pl.ds(c * B + g * 16, 16)]
                for el in range(16):
                    e = g * 16 + el
                    w = wvec[el]
                    for j in range(D // 16):
                        sl = pl.ds(j * 16, 16)
                        rows[i][e, sl] = rows[i][e, sl] * w

        for i in range(NBUF):
            gather(i, i).start()

        def chunk_body(k, carry):
            for i in range(NBUF):
                c = k * NBUF + i
                gather(c, i).wait()
                scale(c, i)
                scatter(c, i).start(add=True)
            for i in range(NBUF):
                c2 = k * NBUF + i + NBUF
                scatter(c2, i).wait()

                @pl.when(c2 < NCHUNK)
                def _():
                    gather(c2, i).start()
            return carry

        lax.fori_loop(0, NCHUNK // NBUF, chunk_body, 0)

        plsc.subcore_barrier()

        for i in range(8):
            cz = sid * 8 + i

            @pl.when(cz < NCHUNK)
            def _():
                sl = pl.ds(cz * RB, RB)

                @pl.when(cid == 0)
                def _():
                    pltpu.sync_copy(acc.at[sl], p0_hbm.at[sl])

                @pl.when(cid == 1)
                def _():
                    pltpu.sync_copy(acc.at[sl], p1_hbm.at[sl])

    return spmm


_spmm_h = _make_spmm(H)
_spmm_c = _make_spmm(CP)


def kernel(x, edge_index, edge_weight, W1, b1, W2, b2):
    dst = edge_index[0].reshape(NW, NCHUNK, B)
    src = edge_index[1].reshape(NW, EW)
    w = edge_weight.reshape(NW, EW)

    h = pl.pallas_call(
        _mm1_body,
        out_shape=jax.ShapeDtypeStruct((N, H), jnp.float32),
    )(x, W1.T, b1.reshape(1, H))

    p0, p1 = _spmm_h(h, src, dst, w)

    w2p = jnp.pad(W2.T, ((0, 0), (0, CP - C)))
    b2p = jnp.pad(b2, (0, CP - C)).reshape(1, CP)
    h2 = pl.pallas_call(
        _mm2_body,
        out_shape=jax.ShapeDtypeStruct((N, CP), jnp.float32),
    )(p0, p1, w2p, b2p)

    q0, q1 = _spmm_c(h2, src, dst, w)

    out = pl.pallas_call(
        _lsm_body,
        out_shape=jax.ShapeDtypeStruct((N, C), jnp.float32),
    )(q0, q1)
    return out


# speedup vs baseline: 1.2112x; 1.2112x over previous
import functools

import jax
import jax.numpy as jnp
from jax import lax
from jax.experimental import pallas as pl
from jax.experimental.pallas import tpu as pltpu
from jax.experimental.pallas import tpu_sc as plsc

N = 10000
E = 320000
F_IN = 128
H = 64
C = 40
CP = 48

NC = 2
NS = 16
NW = NC * NS
EW = E // NW
B = 80
NCHUNK = EW // B
RB = 80
NBUF = 4


def _mm1_body(x_ref, w_ref, b_ref, o_ref):
    o_ref[...] = (
        jnp.dot(x_ref[...], w_ref[...], preferred_element_type=jnp.float32)
        + b_ref[...]
    )


def _mm2_body(p0_ref, p1_ref, w_ref, b_ref, o_ref):
    h = jnp.maximum(p0_ref[...] + p1_ref[...], 0.0)
    o_ref[...] = (
        jnp.dot(h, w_ref[...], preferred_element_type=jnp.float32) + b_ref[...]
    )


def _lsm_body(q0_ref, q1_ref, o_ref):
    logits = (q0_ref[...] + q1_ref[...])[:, :C]
    m = jnp.max(logits, axis=1, keepdims=True)
    ex = jnp.exp(logits - m)
    lse = jnp.log(jnp.sum(ex, axis=1, keepdims=True)) + m
    o_ref[...] = logits - lse


def _make_spmm(D):
    mesh = plsc.VectorSubcoreMesh(core_axis_name="c", subcore_axis_name="s")

    @functools.partial(
        pl.kernel,
        out_type=[
            jax.ShapeDtypeStruct((N, D), jnp.float32),
            jax.ShapeDtypeStruct((N, D), jnp.float32),
        ],
        mesh=mesh,
        scratch_types=[
            pltpu.VMEM((EW,), jnp.int32),
            pltpu.VMEM((NCHUNK, B), jnp.int32),
            pltpu.VMEM((EW,), jnp.float32),
            [pltpu.VMEM((B, D), jnp.float32)] * NBUF,
            pltpu.VMEM_SHARED((N, D), jnp.float32),
            [pltpu.SemaphoreType.DMA] * NBUF,
            [pltpu.SemaphoreType.DMA] * NBUF,
        ],
        compiler_params=pltpu.CompilerParams(use_tc_tiling_on_sc=False),
    )
    def spmm(h_hbm, src_hbm, dst_hbm, w_hbm, p0_hbm, p1_hbm,
             srcv, dstv, wv, rows, acc, gsem, ssem):
        cid = lax.axis_index("c")
        sid = lax.axis_index("s")
        wid = sid * NC + cid

        zero16 = jnp.zeros((16,), jnp.float32)
        for e in range(B):
            for j in range(D // 16):
                rows[0][e, pl.ds(j * 16, 16)] = zero16
        for i in range(8):
            cz = sid * 8 + i

            @pl.when(cz < NCHUNK)
            def _():
                pltpu.sync_copy(rows[0], acc.at[pl.ds(cz * RB, RB)])

        pltpu.sync_copy(src_hbm.at[wid], srcv)
        pltpu.sync_copy(dst_hbm.at[wid], dstv)
        pltpu.sync_copy(w_hbm.at[wid], wv)

        plsc.subcore_barrier()

        def gather(c, i):
            return pltpu.make_async_copy(
                h_hbm.at[srcv.at[pl.ds(c * B, B)]], rows[i], gsem[i])

        def scatter(c, i):
            return pltpu.make_async_copy(rows[i], acc.at[dstv.at[c]], ssem[i])

        def scale(c, i):
            for g in range(B // 16):
                wvec = wv[pl.ds(c * B + g * 16, 16)]
                for el in range(16):
                    e = g * 16 + el
                    w = wvec[el]
                    for j in range(D // 16):
                        sl = pl.ds(j * 16, 16)
                        rows[i][e, sl] = rows[i][e, sl] * w

        for i in range(NBUF):
            gather(i, i).start()

        def chunk_body(k, carry):
            for i in range(NBUF):
                c = k * NBUF + i
                gather(c, i).wait()
                scale(c, i)
                scatter(c, i).start(add=True)
            for i in range(NBUF):
                c2 = k * NBUF + i + NBUF
                scatter(c2, i).wait()

                @pl.when(c2 < NCHUNK)
                def _():
                    gather(c2, i).start()
            return carry

        lax.fori_loop(0, NCHUNK // NBUF, chunk_body, 0)

        ct = (NCHUNK // NBUF) * NBUF
        gather(ct, 0).wait()
        scale(ct, 0)
        scatter(ct, 0).start(add=True)
        scatter(ct, 0).wait()

        plsc.subcore_barrier()

        for i in range(8):
            cz = sid * 8 + i

            @pl.when(cz < NCHUNK)
            def _():
                sl = pl.ds(cz * RB, RB)

                @pl.when(cid == 0)
                def _():
                    pltpu.sync_copy(acc.at[sl], p0_hbm.at[sl])

                @pl.when(cid == 1)
                def _():
                    pltpu.sync_copy(acc.at[sl], p1_hbm.at[sl])

    return spmm


_spmm_h = _make_spmm(H)
_spmm_c = _make_spmm(CP)


def kernel(x, edge_index, edge_weight, W1, b1, W2, b2):
    dst = edge_index[0].reshape(NW, NCHUNK, B)
    src = edge_index[1].reshape(NW, EW)
    w = edge_weight.reshape(NW, EW)

    h = pl.pallas_call(
        _mm1_body,
        out_shape=jax.ShapeDtypeStruct((N, H), jnp.float32),
    )(x, W1.T, b1.reshape(1, H))

    p0, p1 = _spmm_h(h, src, dst, w)

    w2p = jnp.pad(W2.T, ((0, 0), (0, CP - C)))
    b2p = jnp.pad(b2, (0, CP - C)).reshape(1, CP)
    h2 = pl.pallas_call(
        _mm2_body,
        out_shape=jax.ShapeDtypeStruct((N, CP), jnp.float32),
    )(p0, p1, w2p, b2p)

    q0, q1 = _spmm_c(h2, src, dst, w)

    out = pl.pallas_call(
        _lsm_body,
        out_shape=jax.ShapeDtypeStruct((N, C), jnp.float32),
    )(q0, q1)
    return out
